# baseline (speedup 1.0000x reference)
import functools

import jax
import jax.numpy as jnp
from jax import lax
from jax.experimental import pallas as pl
from jax.experimental.pallas import tpu as pltpu
from jax.experimental.pallas import tpu_sc as plsc

D_EMBED = 128
N_VOCAB = 101
BATCH = 4096
SEQ = 200
CHUNK = 100
N_CHUNK = SEQ // CHUNK
G = 1


def _sc_embed(x2, embed, cls_row):
    info = plsc.get_sparse_core_info()
    nw = info.num_cores * info.num_subcores
    nb = BATCH // nw

    mesh = plsc.VectorSubcoreMesh(core_axis_name="c", subcore_axis_name="s")

    @functools.partial(
        pl.kernel,
        out_type=jax.ShapeDtypeStruct((BATCH, SEQ + 1, D_EMBED), jnp.float32),
        mesh=mesh,
        scratch_types=[
            pltpu.VMEM((nb * N_CHUNK, CHUNK), jnp.int32),
            pltpu.VMEM((G, SEQ + 1, D_EMBED), jnp.float32),
            pltpu.VMEM((G, SEQ + 1, D_EMBED), jnp.float32),
            pltpu.SemaphoreType.DMA,
            pltpu.SemaphoreType.DMA,
            pltpu.SemaphoreType.DMA,
        ],
    )
    def k(x_hbm, tab_hbm, cls_hbm, out_hbm, idx_v, buf_a, buf_b,
          gsem, osem_a, osem_b):
        sid = lax.axis_index("s")
        wid = sid * info.num_cores + lax.axis_index("c")

        for buf in (buf_a, buf_b):
            for g in range(G):
                pltpu.sync_copy(cls_hbm, buf.at[g, pl.ds(0, 1)])
        pltpu.sync_copy(x_hbm.at[pl.ds(wid * (nb * N_CHUNK), nb * N_CHUNK)], idx_v)

        def gather(blk, buf):
            cps = []
            for g in range(G):
                for c in range(N_CHUNK):
                    cps.append(pltpu.async_copy(
                        tab_hbm.at[idx_v.at[(blk * G + g) * N_CHUNK + c]],
                        buf.at[g, pl.ds(1 + c * CHUNK, CHUNK)],
                        gsem))
            for cp in cps:
                cp.wait()

        def put(blk, buf, sem):
            return pltpu.async_copy(
                buf, out_hbm.at[pl.ds(wid * nb + blk * G, G)], sem)

        nblk = nb // G
        gather(0, buf_a)

        def body(i, carry):
            wa = put(2 * i, buf_a, osem_a)
            gather(2 * i + 1, buf_b)
            wb = put(2 * i + 1, buf_b, osem_b)
            wa.wait()

            @pl.when(i + 1 < nblk // 2)
            def _():
                gather(2 * i + 2, buf_a)
            wb.wait()
            return carry

        lax.fori_loop(0, nblk // 2, body, 0)

    return k(x2, embed, cls_row)


def kernel(x, embed, first_cls):
    x2 = x.astype(jnp.int32).reshape(BATCH * N_CHUNK, CHUNK)
    cls_row = first_cls.reshape(1, D_EMBED)
    return _sc_embed(x2, embed, cls_row)
